# baseline (speedup 1.0000x reference)
import functools

import jax
import jax.numpy as jnp
from jax import lax
from jax.experimental import pallas as pl
from jax.experimental.pallas import tpu as pltpu, tpu_sc as plsc

N = 10000
D = 128
E = 320000
H = 128
C = 10

HA = 144

NC, NS, L = 2, 16, 16
NW = NC * NS
EW = E // NW
CH = 64
CPW = 160
NCHK = NW * CPW
EPAD = NCHK * CH - E
NP = N + 16
CHZ = 128
NRCH = N // CHZ
NRT = N - NRCH * CHZ

@functools.cache
def _mesh():
    return plsc.VectorSubcoreMesh(core_axis_name="c", subcore_axis_name="s")


def _haug_body(x_ref, w_ref, b_ref, out_ref):
    h = jnp.maximum(jnp.dot(x_ref[...], w_ref[...],
                            preferred_element_type=jnp.float32)
                    + b_ref[...][None, :], 0.0)
    pad = (lax.broadcasted_iota(jnp.int32, (h.shape[0], HA - H), 1) == 0)
    out_ref[...] = jnp.concatenate([h, pad.astype(jnp.float32)], axis=1)


_BLK = 1000

_haug = pl.pallas_call(
    _haug_body,
    grid=(N // _BLK,),
    in_specs=[
        pl.BlockSpec((_BLK, D), lambda i: (i, 0)),
        pl.BlockSpec((D, H), lambda i: (0, 0)),
        pl.BlockSpec((H,), lambda i: (0,)),
    ],
    out_specs=pl.BlockSpec((_BLK, HA), lambda i: (i, 0)),
    out_shape=jax.ShapeDtypeStruct((N, HA), jnp.float32),
)


KG = 4


def _agg_body(h_hbm, src_hbm, dst_hbm, zeros_hbm, out_hbm,
              src0_v, src1_v, src2_v, src3_v, dst0_v, dst1_v, dst2_v, dst3_v,
              rows0_v, rows1_v, rows2_v, rows3_v, agg_sh,
              gs0, gs1, gs2, gs3, is0, is1, is2, is3,
              id0, id1, id2, id3, ss0, ss1, ss2, ss3):
    c = lax.axis_index("c")
    s = lax.axis_index("s")
    wid = c * NS + s
    srcs = (src0_v, src1_v, src2_v, src3_v)
    dsts = (dst0_v, dst1_v, dst2_v, dst3_v)
    rows = (rows0_v, rows1_v, rows2_v, rows3_v)
    gsem = (gs0, gs1, gs2, gs3)
    isem = (is0, is1, is2, is3)
    idsem = (id0, id1, id2, id3)
    ssem = (ss0, ss1, ss2, ss3)

    for j in range((NRCH + NS - 1) // NS):
        idx = j * NS + s

        @pl.when(idx < NRCH)
        def _():
            pltpu.sync_copy(zeros_hbm, agg_sh.at[pl.ds(idx * CHZ, CHZ)])

    @pl.when(s == 0)
    def _():
        pltpu.sync_copy(zeros_hbm.at[pl.ds(0, NRT)],
                        agg_sh.at[pl.ds(NRCH * CHZ, NRT)])

    plsc.subcore_barrier()

    base = wid * CPW * CH

    def step(jj, carry):
        jb = base + jj * KG * CH
        for i in range(KG):
            off = jb + i * CH
            pltpu.sync_copy(src_hbm.at[pl.ds(off, CH)], srcs[i])
            pltpu.sync_copy(dst_hbm.at[pl.ds(off, CH)], dsts[i])
        for i in range(KG):
            pltpu.async_copy(h_hbm.at[srcs[i]], rows[i], gsem[i]).wait()
            pltpu.sync_copy(rows[i], agg_sh.at[dsts[i]], add=True)
        return carry

    lax.fori_loop(0, CPW // KG, step, 0)

    plsc.subcore_barrier()

    for j in range((NRCH + NS - 1) // NS):
        idx = j * NS + s

        @pl.when(idx < NRCH)
        def _():
            pltpu.sync_copy(agg_sh.at[pl.ds(idx * CH, CH)],
                            out_hbm.at[c, pl.ds(idx * CH, CH)])

    @pl.when(s == 0)
    def _():
        pltpu.sync_copy(agg_sh.at[pl.ds(NRCH * CH, NRT)],
                        out_hbm.at[c, pl.ds(NRCH * CH, NRT)])


@functools.cache
def _agg():
    return pl.kernel(
        _agg_body,
        out_type=jax.ShapeDtypeStruct((NC, N, HA), jnp.float32),
        mesh=_mesh(),
        scratch_types=(
            [pltpu.VMEM((CH,), jnp.int32)] * 8
            + [pltpu.VMEM((CH, HA), jnp.float32)] * 4
            + [pltpu.VMEM_SHARED((NP, HA), jnp.float32)]
            + [pltpu.SemaphoreType.DMA] * 16
        ),
        compiler_params=pltpu.CompilerParams(use_tc_tiling_on_sc=False),
    )


def _post_body(aggs_ref, wet_ref, wc_ref, bc_ref, s2_ref, pc_ref, acc_ref):
    i = pl.program_id(0)
    a = aggs_ref[0] + aggs_ref[1]
    deg = jnp.maximum(a[:, H:H + 1], 1.0)
    z = a[:, :H] / deg
    st = jnp.sum(z * wet_ref[0:1, :], axis=1, keepdims=True)
    sb = jnp.sum(z * wet_ref[1:2, :], axis=1, keepdims=True)
    s2_ref[...] = jnp.concatenate([st, sb], axis=1)

    @pl.when(i == 0)
    def _():
        acc_ref[...] = jnp.zeros_like(acc_ref)

    acc_ref[...] += jnp.sum(z, axis=0, keepdims=True)

    @pl.when(i == N // _BLK - 1)
    def _():
        zg = acc_ref[...] / jnp.float32(N)
        pc_ref[...] = jnp.dot(zg, wc_ref[...],
                              preferred_element_type=jnp.float32) + bc_ref[...]


_post = pl.pallas_call(
    _post_body,
    grid=(N // _BLK,),
    in_specs=[
        pl.BlockSpec((NC, _BLK, HA), lambda i: (0, i, 0)),
        pl.BlockSpec((2, H), lambda i: (0, 0)),
        pl.BlockSpec((H, C), lambda i: (0, 0)),
        pl.BlockSpec((1, C), lambda i: (0, 0)),
    ],
    out_specs=[
        pl.BlockSpec((_BLK, 2), lambda i: (i, 0)),
        pl.BlockSpec((1, C), lambda i: (0, 0)),
    ],
    out_shape=[
        jax.ShapeDtypeStruct((N, 2), jnp.float32),
        jax.ShapeDtypeStruct((1, C), jnp.float32),
    ],
    scratch_shapes=[pltpu.VMEM((1, H), jnp.float32)],
)


def _edge_body(s2_hbm, src_hbm, dst_hbm, be_hbm, out_hbm,
               s2_v, src_v, dst_v, out_v, be_v):
    c = lax.axis_index("c")
    s = lax.axis_index("s")
    wid = c * NS + s
    base = wid * EW

    pltpu.sync_copy(s2_hbm, s2_v)
    pltpu.sync_copy(src_hbm.at[pl.ds(base, EW)], src_v)
    pltpu.sync_copy(dst_hbm.at[pl.ds(base, EW)], dst_v)
    pltpu.sync_copy(be_hbm, be_v)
    bev = be_v[...]
    col0 = jnp.zeros((L,), jnp.int32)
    col1 = col0 + 1

    def body(i, carry):
        ids_s = src_v[pl.ds(i * L, L)]
        ids_d = dst_v[pl.ds(i * L, L)]
        vs = plsc.load_gather(s2_v, [ids_s, col0])
        vd = plsc.load_gather(s2_v, [ids_d, col1])
        out_v[pl.ds(i * L, L)] = vs + vd + bev
        return carry

    lax.fori_loop(0, EW // L, body, 0)
    pltpu.sync_copy(out_v, out_hbm.at[pl.ds(base, EW)])


@functools.cache
def _edge():
    return pl.kernel(
        _edge_body,
        out_type=jax.ShapeDtypeStruct((E,), jnp.float32),
        mesh=_mesh(),
        scratch_types=[
            pltpu.VMEM((N, 2), jnp.float32),
            pltpu.VMEM((EW,), jnp.int32),
            pltpu.VMEM((EW,), jnp.int32),
            pltpu.VMEM((EW,), jnp.float32),
            pltpu.VMEM((L,), jnp.float32),
        ],
        compiler_params=pltpu.CompilerParams(use_tc_tiling_on_sc=False,
                                             needs_layout_passes=False),
    )


def kernel(x, edge_index, W1, b1, Wc, bc, We, be):
    src = edge_index[0].astype(jnp.int32)
    dst = edge_index[1].astype(jnp.int32)

    h_aug = _haug(x, W1, b1)
    zeros_blk = jnp.zeros((CHZ, HA), jnp.float32)
    src_p = jnp.concatenate([src, jnp.zeros((EPAD,), jnp.int32)])
    dst_p = jnp.concatenate([dst, jnp.full((EPAD,), N, jnp.int32)])
    aggs = _agg()(h_aug, src_p, dst_p, zeros_blk)

    s2, pred_class = _post(aggs, We.reshape(2, H), Wc, bc.reshape(1, C))

    be16 = jnp.broadcast_to(be, (L,)).astype(jnp.float32)
    pe = _edge()(s2, src, dst, be16)
    return (pe.reshape(E, 1), pred_class)


# speedup vs baseline: 2.1872x; 2.1872x over previous
import functools

import jax
import jax.numpy as jnp
from jax import lax
from jax.experimental import pallas as pl
from jax.experimental.pallas import tpu as pltpu, tpu_sc as plsc

N = 10000
D = 128
E = 320000
H = 128
C = 10

HA = 144

NC, NS, L = 2, 16, 16
NW = NC * NS
EW = E // NW
CH = 128
NFULL = EW // CH
TAIL = EW - NFULL * CH
CHZ = 128
NRCH = N // CHZ
NRT = N - NRCH * CHZ

@functools.cache
def _mesh():
    return plsc.VectorSubcoreMesh(core_axis_name="c", subcore_axis_name="s")


def _haug_body(x_ref, w_ref, b_ref, out_ref):
    h = jnp.maximum(jnp.dot(x_ref[...], w_ref[...],
                            preferred_element_type=jnp.float32)
                    + b_ref[...][None, :], 0.0)
    pad = (lax.broadcasted_iota(jnp.int32, (h.shape[0], HA - H), 1) == 0)
    out_ref[...] = jnp.concatenate([h, pad.astype(jnp.float32)], axis=1)


_BLK = 1000

_haug = pl.pallas_call(
    _haug_body,
    grid=(N // _BLK,),
    in_specs=[
        pl.BlockSpec((_BLK, D), lambda i: (i, 0)),
        pl.BlockSpec((D, H), lambda i: (0, 0)),
        pl.BlockSpec((H,), lambda i: (0,)),
    ],
    out_specs=pl.BlockSpec((_BLK, HA), lambda i: (i, 0)),
    out_shape=jax.ShapeDtypeStruct((N, HA), jnp.float32),
)


def _agg_body(h_hbm, src_hbm, dst_hbm, zeros_hbm, out_hbm,
              src_v, dst_v, src1_v, dst1_v, rows_v, rows1_v,
              src_t, dst_t, rows_t, agg_sh, sem, sem1):
    c = lax.axis_index("c")
    s = lax.axis_index("s")
    wid = c * NS + s

    for j in range((NRCH + NS - 1) // NS):
        idx = j * NS + s

        @pl.when(idx < NRCH)
        def _():
            pltpu.sync_copy(zeros_hbm, agg_sh.at[pl.ds(idx * CHZ, CHZ)])

    @pl.when(s == 0)
    def _():
        pltpu.sync_copy(zeros_hbm.at[pl.ds(0, NRT)],
                        agg_sh.at[pl.ds(NRCH * CHZ, NRT)])

    plsc.subcore_barrier()

    base = wid * EW

    def body(jj, carry):
        off = base + jj * 2 * CH
        pltpu.sync_copy(src_hbm.at[pl.ds(off, CH)], src_v)
        pltpu.sync_copy(dst_hbm.at[pl.ds(off, CH)], dst_v)
        pltpu.sync_copy(src_hbm.at[pl.ds(off + CH, CH)], src1_v)
        pltpu.sync_copy(dst_hbm.at[pl.ds(off + CH, CH)], dst1_v)
        g0 = pltpu.async_copy(h_hbm.at[src_v], rows_v, sem)
        g1 = pltpu.async_copy(h_hbm.at[src1_v], rows1_v, sem1)
        g0.wait()
        pltpu.sync_copy(rows_v, agg_sh.at[dst_v], add=True)
        g1.wait()
        pltpu.sync_copy(rows1_v, agg_sh.at[dst1_v], add=True)
        return carry

    lax.fori_loop(0, NFULL // 2, body, 0)

    offt = base + NFULL * CH
    pltpu.sync_copy(src_hbm.at[pl.ds(offt, TAIL)], src_t)
    pltpu.sync_copy(dst_hbm.at[pl.ds(offt, TAIL)], dst_t)
    pltpu.async_copy(h_hbm.at[src_t], rows_t, sem).wait()
    pltpu.sync_copy(rows_t, agg_sh.at[dst_t], add=True)
    plsc.subcore_barrier()

    for j in range((NRCH + NS - 1) // NS):
        idx = j * NS + s

        @pl.when(idx < NRCH)
        def _():
            pltpu.sync_copy(agg_sh.at[pl.ds(idx * CHZ, CHZ)],
                            out_hbm.at[c, pl.ds(idx * CHZ, CHZ)])

    @pl.when(s == 0)
    def _():
        pltpu.sync_copy(agg_sh.at[pl.ds(NRCH * CHZ, NRT)],
                        out_hbm.at[c, pl.ds(NRCH * CHZ, NRT)])


@functools.cache
def _agg():
    return pl.kernel(
        _agg_body,
        out_type=jax.ShapeDtypeStruct((NC, N, HA), jnp.float32),
        mesh=_mesh(),
        scratch_types=[
            pltpu.VMEM((CH,), jnp.int32),
            pltpu.VMEM((CH,), jnp.int32),
            pltpu.VMEM((CH,), jnp.int32),
            pltpu.VMEM((CH,), jnp.int32),
            pltpu.VMEM((CH, HA), jnp.float32),
            pltpu.VMEM((CH, HA), jnp.float32),
            pltpu.VMEM((TAIL,), jnp.int32),
            pltpu.VMEM((TAIL,), jnp.int32),
            pltpu.VMEM((TAIL, HA), jnp.float32),
            pltpu.VMEM_SHARED((N, HA), jnp.float32),
            pltpu.SemaphoreType.DMA,
            pltpu.SemaphoreType.DMA,
        ],
        compiler_params=pltpu.CompilerParams(use_tc_tiling_on_sc=False),
    )


def _post_body(aggs_ref, wet_ref, wc_ref, bc_ref, s2_ref, pc_ref, acc_ref):
    i = pl.program_id(0)
    a = aggs_ref[0] + aggs_ref[1]
    deg = jnp.maximum(a[:, H:H + 1], 1.0)
    z = a[:, :H] / deg
    st = jnp.sum(z * wet_ref[0:1, :], axis=1, keepdims=True)
    sb = jnp.sum(z * wet_ref[1:2, :], axis=1, keepdims=True)
    s2_ref[...] = jnp.concatenate([st, sb], axis=1)

    @pl.when(i == 0)
    def _():
        acc_ref[...] = jnp.zeros_like(acc_ref)

    acc_ref[...] += jnp.sum(z, axis=0, keepdims=True)

    @pl.when(i == N // _BLK - 1)
    def _():
        zg = acc_ref[...] / jnp.float32(N)
        pc_ref[...] = jnp.dot(zg, wc_ref[...],
                              preferred_element_type=jnp.float32) + bc_ref[...]


_post = pl.pallas_call(
    _post_body,
    grid=(N // _BLK,),
    in_specs=[
        pl.BlockSpec((NC, _BLK, HA), lambda i: (0, i, 0)),
        pl.BlockSpec((2, H), lambda i: (0, 0)),
        pl.BlockSpec((H, C), lambda i: (0, 0)),
        pl.BlockSpec((1, C), lambda i: (0, 0)),
    ],
    out_specs=[
        pl.BlockSpec((_BLK, 2), lambda i: (i, 0)),
        pl.BlockSpec((1, C), lambda i: (0, 0)),
    ],
    out_shape=[
        jax.ShapeDtypeStruct((N, 2), jnp.float32),
        jax.ShapeDtypeStruct((1, C), jnp.float32),
    ],
    scratch_shapes=[pltpu.VMEM((1, H), jnp.float32)],
)


def _edge_body(s2_hbm, src_hbm, dst_hbm, be_hbm, out_hbm,
               s2_v, src_v, dst_v, out_v, be_v):
    c = lax.axis_index("c")
    s = lax.axis_index("s")
    wid = c * NS + s
    base = wid * EW

    pltpu.sync_copy(s2_hbm, s2_v)
    pltpu.sync_copy(src_hbm.at[pl.ds(base, EW)], src_v)
    pltpu.sync_copy(dst_hbm.at[pl.ds(base, EW)], dst_v)
    pltpu.sync_copy(be_hbm, be_v)
    bev = be_v[...]
    col0 = jnp.zeros((L,), jnp.int32)
    col1 = col0 + 1

    def body(i, carry):
        ids_s = src_v[pl.ds(i * L, L)]
        ids_d = dst_v[pl.ds(i * L, L)]
        vs = plsc.load_gather(s2_v, [ids_s, col0])
        vd = plsc.load_gather(s2_v, [ids_d, col1])
        out_v[pl.ds(i * L, L)] = vs + vd + bev
        return carry

    lax.fori_loop(0, EW // L, body, 0)
    pltpu.sync_copy(out_v, out_hbm.at[pl.ds(base, EW)])


@functools.cache
def _edge():
    return pl.kernel(
        _edge_body,
        out_type=jax.ShapeDtypeStruct((E,), jnp.float32),
        mesh=_mesh(),
        scratch_types=[
            pltpu.VMEM((N, 2), jnp.float32),
            pltpu.VMEM((EW,), jnp.int32),
            pltpu.VMEM((EW,), jnp.int32),
            pltpu.VMEM((EW,), jnp.float32),
            pltpu.VMEM((L,), jnp.float32),
        ],
        compiler_params=pltpu.CompilerParams(use_tc_tiling_on_sc=False,
                                             needs_layout_passes=False),
    )


def kernel(x, edge_index, W1, b1, Wc, bc, We, be):
    src = edge_index[0].astype(jnp.int32)
    dst = edge_index[1].astype(jnp.int32)

    h_aug = _haug(x, W1, b1)
    zeros_blk = jnp.zeros((CHZ, HA), jnp.float32)
    aggs = _agg()(h_aug, src, dst, zeros_blk)

    s2, pred_class = _post(aggs, We.reshape(2, H), Wc, bc.reshape(1, C))

    be16 = jnp.broadcast_to(be, (L,)).astype(jnp.float32)
    pe = _edge()(s2, src, dst, be16)
    return (pe.reshape(E, 1), pred_class)


# speedup vs baseline: 2.9453x; 1.3466x over previous
import functools

import jax
import jax.numpy as jnp
from jax import lax
from jax.experimental import pallas as pl
from jax.experimental.pallas import tpu as pltpu, tpu_sc as plsc

N = 10000
D = 128
E = 320000
H = 128
C = 10

HA = 144

NC, NS, L = 2, 16, 16
NW = NC * NS
EW = E // NW
CH = 128
NFULL = EW // CH
TAIL = EW - NFULL * CH
CHZ = 128
NRCH = N // CHZ
NRT = N - NRCH * CHZ

@functools.cache
def _mesh():
    return plsc.VectorSubcoreMesh(core_axis_name="c", subcore_axis_name="s")


def _haug_body(x_ref, w_ref, b_ref, out_ref):
    h = jnp.maximum(jnp.dot(x_ref[...], w_ref[...],
                            preferred_element_type=jnp.float32)
                    + b_ref[...][None, :], 0.0)
    pad = (lax.broadcasted_iota(jnp.int32, (h.shape[0], HA - H), 1) == 0)
    out_ref[...] = jnp.concatenate([h, pad.astype(jnp.float32)], axis=1)


_BLK = 1000

_haug = pl.pallas_call(
    _haug_body,
    grid=(N // _BLK,),
    in_specs=[
        pl.BlockSpec((_BLK, D), lambda i: (i, 0)),
        pl.BlockSpec((D, H), lambda i: (0, 0)),
        pl.BlockSpec((H,), lambda i: (0,)),
    ],
    out_specs=pl.BlockSpec((_BLK, HA), lambda i: (i, 0)),
    out_shape=jax.ShapeDtypeStruct((N, HA), jnp.float32),
)


GRP = 13


def _agg_body(h_hbm, src_hbm, dst_hbm, zeros_hbm, out_hbm,
              src_v, dst_v, src1_v, dst1_v, rows_v, rows1_v,
              src_t, dst_t, rows_t, agg_sh, sem, sem1,
              sem_i0, sem_i1, sem_d0, sem_d1):
    c = lax.axis_index("c")
    s = lax.axis_index("s")
    wid = c * NS + s

    for j in range((NRCH + NS - 1) // NS):
        idx = j * NS + s

        @pl.when(idx < NRCH)
        def _():
            pltpu.sync_copy(zeros_hbm, agg_sh.at[pl.ds(idx * CHZ, CHZ)])

    @pl.when(s == 0)
    def _():
        pltpu.sync_copy(zeros_hbm.at[pl.ds(0, NRT)],
                        agg_sh.at[pl.ds(NRCH * CHZ, NRT)])

    plsc.subcore_barrier()

    base = wid * EW

    srcb = (src_v, src1_v)
    dstb = (dst_v, dst1_v)
    rows = (rows_v, rows1_v)
    isem = (sem_i0, sem_i1)
    idsem = (sem_d0, sem_d1)
    gsem = (sem, sem1)

    def idx_fire(off, b):
        return (pltpu.async_copy(src_hbm.at[pl.ds(off, CH)], srcb[b],
                                 isem[b]),
                pltpu.async_copy(dst_hbm.at[pl.ds(off, CH)], dstb[b],
                                 idsem[b]))

    def body(jj, carry):
        gb = base + jj * GRP * CH
        ih = [None, None]
        gh = [None, None]
        ih[0] = idx_fire(gb, 0)
        ih[0][0].wait()
        gh[0] = pltpu.async_copy(h_hbm.at[srcb[0]], rows[0], gsem[0])
        for i in range(GRP):
            b = i % 2
            nb = 1 - b
            if i + 1 < GRP:
                ih[nb] = idx_fire(gb + (i + 1) * CH, nb)
            gh[b].wait()
            if i + 1 < GRP:
                ih[nb][0].wait()
                gh[nb] = pltpu.async_copy(h_hbm.at[srcb[nb]], rows[nb],
                                          gsem[nb])
            ih[b][1].wait()
            pltpu.sync_copy(rows[b], agg_sh.at[dstb[b]], add=True)
        return carry

    lax.fori_loop(0, NFULL // GRP, body, 0)

    offt = base + NFULL * CH
    pltpu.sync_copy(src_hbm.at[pl.ds(offt, TAIL)], src_t)
    pltpu.sync_copy(dst_hbm.at[pl.ds(offt, TAIL)], dst_t)
    pltpu.async_copy(h_hbm.at[src_t], rows_t, sem).wait()
    pltpu.sync_copy(rows_t, agg_sh.at[dst_t], add=True)
    plsc.subcore_barrier()

    for j in range((NRCH + NS - 1) // NS):
        idx = j * NS + s

        @pl.when(idx < NRCH)
        def _():
            pltpu.sync_copy(agg_sh.at[pl.ds(idx * CHZ, CHZ)],
                            out_hbm.at[c, pl.ds(idx * CHZ, CHZ)])

    @pl.when(s == 0)
    def _():
        pltpu.sync_copy(agg_sh.at[pl.ds(NRCH * CHZ, NRT)],
                        out_hbm.at[c, pl.ds(NRCH * CHZ, NRT)])


@functools.cache
def _agg():
    return pl.kernel(
        _agg_body,
        out_type=jax.ShapeDtypeStruct((NC, N, HA), jnp.float32),
        mesh=_mesh(),
        scratch_types=[
            pltpu.VMEM((CH,), jnp.int32),
            pltpu.VMEM((CH,), jnp.int32),
            pltpu.VMEM((CH,), jnp.int32),
            pltpu.VMEM((CH,), jnp.int32),
            pltpu.VMEM((CH, HA), jnp.float32),
            pltpu.VMEM((CH, HA), jnp.float32),
            pltpu.VMEM((TAIL,), jnp.int32),
            pltpu.VMEM((TAIL,), jnp.int32),
            pltpu.VMEM((TAIL, HA), jnp.float32),
            pltpu.VMEM_SHARED((N, HA), jnp.float32),
            pltpu.SemaphoreType.DMA,
            pltpu.SemaphoreType.DMA,
            pltpu.SemaphoreType.DMA,
            pltpu.SemaphoreType.DMA,
            pltpu.SemaphoreType.DMA,
            pltpu.SemaphoreType.DMA,
        ],
        compiler_params=pltpu.CompilerParams(use_tc_tiling_on_sc=False),
    )


def _post_body(aggs_ref, wet_ref, wc_ref, bc_ref, s2_ref, pc_ref, acc_ref):
    i = pl.program_id(0)
    a = aggs_ref[0] + aggs_ref[1]
    deg = jnp.maximum(a[:, H:H + 1], 1.0)
    z = a[:, :H] / deg
    st = jnp.sum(z * wet_ref[0:1, :], axis=1, keepdims=True)
    sb = jnp.sum(z * wet_ref[1:2, :], axis=1, keepdims=True)
    s2_ref[...] = jnp.concatenate([st, sb], axis=1)

    @pl.when(i == 0)
    def _():
        acc_ref[...] = jnp.zeros_like(acc_ref)

    acc_ref[...] += jnp.sum(z, axis=0, keepdims=True)

    @pl.when(i == N // _BLK - 1)
    def _():
        zg = acc_ref[...] / jnp.float32(N)
        pc_ref[...] = jnp.dot(zg, wc_ref[...],
                              preferred_element_type=jnp.float32) + bc_ref[...]


_post = pl.pallas_call(
    _post_body,
    grid=(N // _BLK,),
    in_specs=[
        pl.BlockSpec((NC, _BLK, HA), lambda i: (0, i, 0)),
        pl.BlockSpec((2, H), lambda i: (0, 0)),
        pl.BlockSpec((H, C), lambda i: (0, 0)),
        pl.BlockSpec((1, C), lambda i: (0, 0)),
    ],
    out_specs=[
        pl.BlockSpec((_BLK, 2), lambda i: (i, 0)),
        pl.BlockSpec((1, C), lambda i: (0, 0)),
    ],
    out_shape=[
        jax.ShapeDtypeStruct((N, 2), jnp.float32),
        jax.ShapeDtypeStruct((1, C), jnp.float32),
    ],
    scratch_shapes=[pltpu.VMEM((1, H), jnp.float32)],
)


def _edge_body(s2_hbm, src_hbm, dst_hbm, be_hbm, out_hbm,
               s2_v, src_v, dst_v, out_v, be_v):
    c = lax.axis_index("c")
    s = lax.axis_index("s")
    wid = c * NS + s
    base = wid * EW

    pltpu.sync_copy(s2_hbm, s2_v)
    pltpu.sync_copy(src_hbm.at[pl.ds(base, EW)], src_v)
    pltpu.sync_copy(dst_hbm.at[pl.ds(base, EW)], dst_v)
    pltpu.sync_copy(be_hbm, be_v)
    bev = be_v[...]
    col0 = jnp.zeros((L,), jnp.int32)
    col1 = col0 + 1

    def body(i, carry):
        ids_s = src_v[pl.ds(i * L, L)]
        ids_d = dst_v[pl.ds(i * L, L)]
        vs = plsc.load_gather(s2_v, [ids_s, col0])
        vd = plsc.load_gather(s2_v, [ids_d, col1])
        out_v[pl.ds(i * L, L)] = vs + vd + bev
        return carry

    lax.fori_loop(0, EW // L, body, 0)
    pltpu.sync_copy(out_v, out_hbm.at[pl.ds(base, EW)])


@functools.cache
def _edge():
    return pl.kernel(
        _edge_body,
        out_type=jax.ShapeDtypeStruct((E,), jnp.float32),
        mesh=_mesh(),
        scratch_types=[
            pltpu.VMEM((N, 2), jnp.float32),
            pltpu.VMEM((EW,), jnp.int32),
            pltpu.VMEM((EW,), jnp.int32),
            pltpu.VMEM((EW,), jnp.float32),
            pltpu.VMEM((L,), jnp.float32),
        ],
        compiler_params=pltpu.CompilerParams(use_tc_tiling_on_sc=False,
                                             needs_layout_passes=False),
    )


def kernel(x, edge_index, W1, b1, Wc, bc, We, be):
    src = edge_index[0].astype(jnp.int32)
    dst = edge_index[1].astype(jnp.int32)

    h_aug = _haug(x, W1, b1)
    zeros_blk = jnp.zeros((CHZ, HA), jnp.float32)
    aggs = _agg()(h_aug, src, dst, zeros_blk)

    s2, pred_class = _post(aggs, We.reshape(2, H), Wc, bc.reshape(1, C))

    be16 = jnp.broadcast_to(be, (L,)).astype(jnp.float32)
    pe = _edge()(s2, src, dst, be16)
    return (pe.reshape(E, 1), pred_class)
